# baseline (speedup 1.0000x reference)
import functools

import jax
import jax.numpy as jnp
from jax import lax
from jax.experimental import pallas as pl
from jax.experimental.pallas import tpu as pltpu
from jax.experimental.pallas import tpu_sc as plsc

TOPK = 8
SEQ = 8192
FEAT = 128
CHUNK = 16
NSC = 4
SHARDS = 8
SC_R = 512
_NEG = float("-inf")

_NET8 = (
    (0, 1), (2, 3), (4, 5), (6, 7),
    (0, 2), (1, 3), (4, 6), (5, 7),
    (1, 2), (5, 6), (0, 4), (3, 7),
    (1, 5), (2, 6),
    (1, 4), (3, 6),
    (2, 4), (3, 5),
    (3, 4),
)


def _sort8(v):
    v = list(v)
    for a, b in _NET8:
        hi = jnp.maximum(v[a], v[b])
        lo = jnp.minimum(v[a], v[b])
        v[a], v[b] = hi, lo
    return v


def _merge_top8(a, b):
    h = [jnp.maximum(a[j], b[7 - j]) for j in range(8)]
    for d in (4, 2, 1):
        nh = list(h)
        for s in range(0, 8, 2 * d):
            for t in range(s, s + d):
                nh[t] = jnp.maximum(h[t], h[t + d])
                nh[t + d] = jnp.minimum(h[t], h[t + d])
        h = nh
    return h


def _tree_collapse(acc, p):
    while p > 1:
        half = p // 2
        acc = _merge_top8([t[:half] for t in acc], [t[half:] for t in acc])
        p = half
    return acc


def _tc_body(x_ref, o_ref):
    group = 8 * CHUNK

    def body(i, acc):
        blk = x_ref[0, pl.ds(i * 2 * group, 2 * group), :]
        v1 = _sort8(blk[j * CHUNK:(j + 1) * CHUNK, :] for j in range(8))
        v2 = _sort8(blk[(8 + j) * CHUNK:(9 + j) * CHUNK, :] for j in range(8))
        w = _merge_top8(v1, v2)
        return tuple(_merge_top8(list(acc), w))

    init = tuple(jnp.full((CHUNK, FEAT), _NEG, jnp.float32) for _ in range(TOPK))
    acc = list(jax.lax.fori_loop(0, SEQ // (2 * group), body, init))
    top = jnp.concatenate(_tree_collapse(acc, CHUNK), axis=0)
    o_ref[0] = top.T


def _tc_topk(x, nb):
    _, s, f = x.shape
    return pl.pallas_call(
        _tc_body,
        grid=(nb,),
        in_specs=[pl.BlockSpec((1, s, f), lambda i: (i, 0, 0))],
        out_specs=pl.BlockSpec((1, f, TOPK), lambda i: (i, 0, 0)),
        out_shape=jax.ShapeDtypeStruct((nb, f, TOPK), x.dtype),
    )(x)


def _sc_partials(x):
    b_total = x.shape[0]
    shard_rows = SEQ // SHARDS
    mesh = plsc.VectorSubcoreMesh(core_axis_name="c", subcore_axis_name="s")

    @functools.partial(
        pl.kernel,
        mesh=mesh,
        out_type=jax.ShapeDtypeStruct((NSC, SHARDS, TOPK, FEAT), jnp.float32),
        scratch_types=[
            pltpu.VMEM((SC_R, FEAT), jnp.float32),
            pltpu.VMEM((TOPK, FEAT), jnp.float32),
        ],
    )
    def k(x_hbm, out_hbm, buf, obuf):
        wid = lax.axis_index("s") * 2 + lax.axis_index("c")
        bb = wid // SHARDS
        shard = wid % SHARDS
        b = b_total - NSC + bb
        row0 = shard * shard_rows
        neg = jnp.full((16,), _NEG, jnp.float32)

        def chunk_body(c, acc):
            pltpu.sync_copy(
                x_hbm.at[b, pl.ds(row0 + c * SC_R, SC_R), :], buf)
            accs = list(acc)
            for col in range(FEAT // 16):
                sl = pl.ds(col * 16, 16)

                def row_body(i, a):
                    base = i * 16
                    v1 = _sort8(buf[base + j, sl] for j in range(8))
                    v2 = _sort8(buf[base + 8 + j, sl] for j in range(8))
                    w = _merge_top8(v1, v2)
                    return tuple(_merge_top8(list(a), w))

                a = lax.fori_loop(0, SC_R // 16, row_body,
                                  tuple(accs[col * 8:(col + 1) * 8]))
                accs[col * 8:(col + 1) * 8] = list(a)
            return tuple(accs)

        acc = lax.fori_loop(0, shard_rows // SC_R, chunk_body,
                            (neg,) * (8 * (FEAT // 16)))
        for col in range(FEAT // 16):
            for kk in range(TOPK):
                obuf[kk, pl.ds(col * 16, 16)] = acc[col * 8 + kk]
        pltpu.sync_copy(obuf, out_hbm.at[bb, shard])

    return k(x)


def _sc_merge(parts):
    def body(p_ref, o_ref):
        acc = [jnp.concatenate([p_ref[0, sh, j:j + 1, :] for sh in range(SHARDS)],
                               axis=0) for j in range(TOPK)]
        top = jnp.concatenate(_tree_collapse(acc, SHARDS), axis=0)
        o_ref[0] = top.T

    return pl.pallas_call(
        body,
        grid=(NSC,),
        in_specs=[pl.BlockSpec((1, SHARDS, TOPK, FEAT), lambda i: (i, 0, 0, 0))],
        out_specs=pl.BlockSpec((1, FEAT, TOPK), lambda i: (i, 0, 0)),
        out_shape=jax.ShapeDtypeStruct((NSC, FEAT, TOPK), jnp.float32),
    )(parts)


def kernel(x):
    b, s, f = x.shape
    tc_out = _tc_topk(x, b - NSC)
    parts = _sc_partials(x)
    sc_out = _sc_merge(parts)
    out = jnp.concatenate([tc_out, sc_out], axis=0)
    return out.reshape(b, f * TOPK)


# speedup vs baseline: 1.2346x; 1.2346x over previous
import jax
import jax.numpy as jnp
from jax.experimental import pallas as pl

TOPK = 8
SEQ = 8192
FEAT = 128
CHUNK = 16
_NEG = float("-inf")

_NET8 = (
    (0, 1), (2, 3), (4, 5), (6, 7),
    (0, 2), (1, 3), (4, 6), (5, 7),
    (1, 2), (5, 6), (0, 4), (3, 7),
    (1, 5), (2, 6),
    (1, 4), (3, 6),
    (2, 4), (3, 5),
    (3, 4),
)


def _sort8(v):
    v = list(v)
    for a, b in _NET8:
        hi = jnp.maximum(v[a], v[b])
        lo = jnp.minimum(v[a], v[b])
        v[a], v[b] = hi, lo
    return v


def _merge_top8(a, b):
    h = [jnp.maximum(a[j], b[7 - j]) for j in range(8)]
    for d in (4, 2, 1):
        nh = list(h)
        for s in range(0, 8, 2 * d):
            for t in range(s, s + d):
                nh[t] = jnp.maximum(h[t], h[t + d])
                nh[t + d] = jnp.minimum(h[t], h[t + d])
        h = nh
    return h


def _kmax_body(x_ref, o_ref):
    group = 8 * CHUNK

    def body(i, acc):
        blk = x_ref[0, pl.ds(i * 4 * group, 4 * group), :]
        v = [_sort8(blk[(8 * g + j) * CHUNK:(8 * g + j + 1) * CHUNK, :]
                    for j in range(8)) for g in range(4)]
        w = _merge_top8(_merge_top8(v[0], v[1]), _merge_top8(v[2], v[3]))
        return tuple(_merge_top8(list(acc), w))

    init = tuple(jnp.full((CHUNK, FEAT), _NEG, jnp.float32) for _ in range(TOPK))
    acc = list(jax.lax.fori_loop(0, SEQ // (4 * group), body, init))

    p = CHUNK
    while p > 1:
        half = p // 2
        acc = _merge_top8([t[:half] for t in acc], [t[half:] for t in acc])
        p = half
    top = jnp.concatenate(acc, axis=0)
    o_ref[0] = top.T


def kernel(x):
    b, s, f = x.shape
    out = pl.pallas_call(
        _kmax_body,
        grid=(b,),
        in_specs=[pl.BlockSpec((1, s, f), lambda i: (i, 0, 0))],
        out_specs=pl.BlockSpec((1, f, TOPK), lambda i: (i, 0, 0)),
        out_shape=jax.ShapeDtypeStruct((b, f, TOPK), x.dtype),
    )(x)
    return out.reshape(b, f * TOPK)
